# baseline (speedup 1.0000x reference)
import functools

import jax
import jax.numpy as jnp
from jax import lax
from jax.experimental import pallas as pl
from jax.experimental.pallas import tpu as pltpu
from jax.experimental.pallas import tpu_sc as plsc

N = 10000
D = 128
H = 128
C = 10
G = 128
E = 320000

NT = 32
CK = 96
HB = 24
CH = HB * (-(-E // (NT * CK * HB)))
NBLK = CH // HB
ND = 3
EP = NT * CH * CK
NSUB = 16
NP = 10240
RPS = NP // NSUB
BM = 1024
NB = NP // BM

_mesh = plsc.VectorSubcoreMesh(core_axis_name="c", subcore_axis_name="s")


@functools.partial(
    pl.kernel, mesh=_mesh,
    out_type=jax.ShapeDtypeStruct((2, NP, H), jnp.float32),
    scratch_types=[
        pltpu.VMEM((CH, CK), jnp.int32),
        pltpu.VMEM((CK, H), jnp.float32),
        pltpu.VMEM_SHARED((NP, H), jnp.float32),
        pltpu.SemaphoreType.DMA,
    ])
def _hist_sc(dstr_hbm, zeros16_hbm, ones16_hbm, out_hbm, didx, ones_v, hist_sh, sem):
    c = lax.axis_index("c")
    s = lax.axis_index("s")
    wid = s * 2 + c
    pltpu.sync_copy(zeros16_hbm, hist_sh.at[pl.ds(s * RPS, RPS)])
    pltpu.sync_copy(ones16_hbm, ones_v)
    pltpu.sync_copy(dstr_hbm.at[wid], didx)
    plsc.subcore_barrier()

    @pl.loop(0, CH)
    def _(j):
        pltpu.sync_copy(ones_v, hist_sh.at[didx.at[j]], add=True)

    plsc.subcore_barrier()
    pltpu.sync_copy(hist_sh.at[pl.ds(s * RPS, RPS)],
                    out_hbm.at[c, pl.ds(s * RPS, RPS)])


@functools.partial(
    pl.kernel, mesh=_mesh,
    out_type=jax.ShapeDtypeStruct((2, NP, H), jnp.float32),
    scratch_types=[
        pltpu.VMEM((2, HB, CK), jnp.int32),
        pltpu.VMEM((2, HB, CK), jnp.int32),
        pltpu.VMEM((ND, CK, H), jnp.float32),
        pltpu.VMEM_SHARED((NP, H), jnp.float32),
        pltpu.SemaphoreType.DMA,
        pltpu.SemaphoreType.DMA,
        pltpu.SemaphoreType.DMA,
        pltpu.SemaphoreType.DMA,
        pltpu.SemaphoreType.DMA,
    ])
def _agg_sc(y_hbm, srcr_hbm, dstr_hbm, zerosh_hbm, out_hbm,
            sring, dring, gbuf, agg_sh, gs0, gs1, gs2, ssem, dsem):
    c = lax.axis_index("c")
    s = lax.axis_index("s")
    wid = s * 2 + c
    gsems = (gs0, gs1, gs2)
    pltpu.sync_copy(zerosh_hbm, agg_sh.at[pl.ds(s * RPS, RPS)])
    pltpu.sync_copy(srcr_hbm.at[wid, pl.ds(0, HB)], sring.at[0])
    pltpu.sync_copy(dstr_hbm.at[wid, pl.ds(0, HB)], dring.at[0])
    pltpu.async_copy(srcr_hbm.at[wid, pl.ds(HB, HB)], sring.at[1], ssem)
    pltpu.async_copy(dstr_hbm.at[wid, pl.ds(HB, HB)], dring.at[1], dsem)
    plsc.subcore_barrier()

    for blk in range(NBLK):
        r = blk % 2
        if blk > 0:
            pltpu.make_async_copy(srcr_hbm.at[wid, pl.ds(0, HB)],
                                  sring.at[r], ssem).wait()
            pltpu.make_async_copy(dstr_hbm.at[wid, pl.ds(0, HB)],
                                  dring.at[r], dsem).wait()
        for k in range(ND):
            pltpu.async_copy(y_hbm.at[sring.at[r, k]], gbuf.at[k], gsems[k])

        @pl.loop(0, HB, step=ND)
        def _(lj):
            for k in range(ND):
                t = lj + k
                pltpu.make_async_copy(y_hbm.at[sring.at[r, t]], gbuf.at[k],
                                      gsems[k]).wait()
                pltpu.sync_copy(gbuf.at[k], agg_sh.at[dring.at[r, t]],
                                add=True)

                @pl.when(t + ND < HB)
                def _():
                    pltpu.async_copy(y_hbm.at[sring.at[r, t + ND]],
                                     gbuf.at[k], gsems[k])

        if blk + 2 < NBLK:
            pltpu.async_copy(srcr_hbm.at[wid, pl.ds((blk + 2) * HB, HB)],
                             sring.at[r], ssem)
            pltpu.async_copy(dstr_hbm.at[wid, pl.ds((blk + 2) * HB, HB)],
                             dring.at[r], dsem)

    plsc.subcore_barrier()
    pltpu.sync_copy(agg_sh.at[pl.ds(s * RPS, RPS)],
                    out_hbm.at[c, pl.ds(s * RPS, RPS)])


def _mm_body(x_ref, w_ref, o_ref):
    o_ref[...] = jnp.dot(x_ref[...], w_ref[...],
                         preferred_element_type=jnp.float32)


def _matmul(x, w):
    m, k = x.shape
    n = w.shape[1]
    return pl.pallas_call(
        _mm_body,
        grid=(m // BM,),
        in_specs=[pl.BlockSpec((BM, k), lambda i: (i, 0)),
                  pl.BlockSpec((k, n), lambda i: (0, 0))],
        out_specs=pl.BlockSpec((BM, n), lambda i: (i, 0)),
        out_shape=jax.ShapeDtypeStruct((m, n), jnp.float32),
    )(x, w)


def _scale_body(xw_ref, hist_ref, y_ref, dinv_ref):
    deg = hist_ref[0, :, 0:1] + hist_ref[1, :, 0:1] + 1.0
    dinv = lax.rsqrt(deg)
    dinv_ref[...] = dinv
    y_ref[...] = xw_ref[...] * dinv


def _scale(xw, hist):
    return pl.pallas_call(
        _scale_body,
        grid=(NB,),
        in_specs=[pl.BlockSpec((BM, H), lambda i: (i, 0)),
                  pl.BlockSpec((2, BM, H), lambda i: (0, i, 0))],
        out_specs=[pl.BlockSpec((BM, H), lambda i: (i, 0)),
                   pl.BlockSpec((BM, 1), lambda i: (i, 0))],
        out_shape=[jax.ShapeDtypeStruct((NP, H), jnp.float32),
                   jax.ShapeDtypeStruct((NP, 1), jnp.float32)],
    )(xw, hist)


def _mid_body(p_ref, y_ref, dinv_ref, b_ref, w_ref, o_ref):
    dinv = dinv_ref[...]
    h = jnp.maximum(dinv * (p_ref[0] + p_ref[1] + y_ref[...]) + b_ref[...], 0.0)
    o_ref[...] = jnp.dot(h, w_ref[...],
                         preferred_element_type=jnp.float32) * dinv


def _mid(p, y, dinv, b, w):
    return pl.pallas_call(
        _mid_body,
        grid=(NB,),
        in_specs=[pl.BlockSpec((2, BM, H), lambda i: (0, i, 0)),
                  pl.BlockSpec((BM, H), lambda i: (i, 0)),
                  pl.BlockSpec((BM, 1), lambda i: (i, 0)),
                  pl.BlockSpec((1, H), lambda i: (0, 0)),
                  pl.BlockSpec((H, H), lambda i: (0, 0))],
        out_specs=pl.BlockSpec((BM, H), lambda i: (i, 0)),
        out_shape=jax.ShapeDtypeStruct((NP, H), jnp.float32),
    )(p, y, dinv, b, w)


def _final_body(q_ref, y_ref, dinv_ref, b_ref, batch_ref, wl_ref, bl_ref,
                o_ref, summ, cnt):
    i = pl.program_id(0)

    @pl.when(i == 0)
    def _():
        summ[...] = jnp.zeros_like(summ)
        cnt[...] = jnp.zeros_like(cnt)

    dinv = dinv_ref[...]
    h = jnp.maximum(dinv * (q_ref[0] + q_ref[1] + y_ref[...]) + b_ref[...], 0.0)
    gid = lax.broadcasted_iota(jnp.int32, (G, BM), 0)
    oht = (batch_ref[...][None, :] == gid).astype(jnp.float32)
    summ[...] += jnp.dot(oht, h, preferred_element_type=jnp.float32)
    cnt[...] += jnp.sum(oht, axis=1, keepdims=True)

    @pl.when(i == NB - 1)
    def _():
        pooled = summ[...] / jnp.maximum(cnt[...], 1.0)
        logits = jnp.dot(pooled, wl_ref[...],
                         preferred_element_type=jnp.float32) + bl_ref[...]
        m = jnp.max(logits, axis=1, keepdims=True)
        lse = m + jnp.log(jnp.sum(jnp.exp(logits - m), axis=1, keepdims=True))
        o_ref[...] = logits - lse


def _final(q, y, dinv, b, batchp, wl, bl):
    return pl.pallas_call(
        _final_body,
        grid=(NB,),
        in_specs=[pl.BlockSpec((2, BM, H), lambda i: (0, i, 0)),
                  pl.BlockSpec((BM, H), lambda i: (i, 0)),
                  pl.BlockSpec((BM, 1), lambda i: (i, 0)),
                  pl.BlockSpec((1, H), lambda i: (0, 0)),
                  pl.BlockSpec((BM,), lambda i: (i,)),
                  pl.BlockSpec((H, C), lambda i: (0, 0)),
                  pl.BlockSpec((1, C), lambda i: (0, 0))],
        out_specs=pl.BlockSpec((G, C), lambda i: (0, 0)),
        out_shape=jax.ShapeDtypeStruct((G, C), jnp.float32),
        scratch_shapes=[pltpu.VMEM((G, H), jnp.float32),
                        pltpu.VMEM((G, 1), jnp.float32)],
    )(q, y, dinv, b, batchp, wl, bl)


def kernel(x, edge_index, batch, W1, b1, W2, b2, Wlin, blin):
    pad = EP - E
    pad_idx = N + (jnp.arange(pad, dtype=jnp.int32) % (NP - N))
    srcp = jnp.concatenate([edge_index[0], pad_idx]).reshape(NT, CH, CK)
    dstp = jnp.concatenate([edge_index[1], pad_idx]).reshape(NT, CH, CK)
    xp = jnp.pad(x, ((0, NP - N), (0, 0)))
    batchp = jnp.pad(batch, (0, NP - N), constant_values=-1)
    onesh = jnp.ones((CK, H), jnp.float32)
    zerosh = jnp.zeros((RPS, H), jnp.float32)

    hist = _hist_sc(dstp, zerosh, onesh)
    xw1 = _matmul(xp, W1)
    y1, dinv = _scale(xw1, hist)
    p = _agg_sc(y1, srcp, dstp, zerosh)
    y2 = _mid(p, y1, dinv, b1.reshape(1, H), W2)
    q = _agg_sc(y2, srcp, dstp, zerosh)
    return _final(q, y2, dinv, b2.reshape(1, H), batchp, Wlin,
                  blin.reshape(1, C))


# speedup vs baseline: 1.0489x; 1.0489x over previous
import functools

import jax
import jax.numpy as jnp
from jax import lax
from jax.experimental import pallas as pl
from jax.experimental.pallas import tpu as pltpu
from jax.experimental.pallas import tpu_sc as plsc

N = 10000
D = 128
H = 128
C = 10
G = 128
E = 320000

NT = 32
CK = 128
HB = 16
CH = HB * (-(-E // (NT * CK * HB)))
NBLK = CH // HB
EP = NT * CH * CK
NSUB = 16
NP = 10240
RPS = NP // NSUB
BM = 1024
NB = NP // BM

_mesh = plsc.VectorSubcoreMesh(core_axis_name="c", subcore_axis_name="s")


@functools.partial(
    pl.kernel, mesh=_mesh,
    out_type=jax.ShapeDtypeStruct((2, NP, H), jnp.float32),
    scratch_types=[
        pltpu.VMEM((CH, CK), jnp.int32),
        pltpu.VMEM((CK, H), jnp.float32),
        pltpu.VMEM_SHARED((NP, H), jnp.float32),
        pltpu.SemaphoreType.DMA,
    ])
def _hist_sc(dstr_hbm, zeros16_hbm, ones16_hbm, out_hbm, didx, ones_v, hist_sh, sem):
    c = lax.axis_index("c")
    s = lax.axis_index("s")
    wid = s * 2 + c
    pltpu.sync_copy(zeros16_hbm, hist_sh.at[pl.ds(s * RPS, RPS)])
    pltpu.sync_copy(ones16_hbm, ones_v)
    pltpu.sync_copy(dstr_hbm.at[wid], didx)
    plsc.subcore_barrier()

    @pl.loop(0, CH)
    def _(j):
        pltpu.sync_copy(ones_v, hist_sh.at[didx.at[j]], add=True)

    plsc.subcore_barrier()
    pltpu.sync_copy(hist_sh.at[pl.ds(s * RPS, RPS)],
                    out_hbm.at[c, pl.ds(s * RPS, RPS)])


@functools.partial(
    pl.kernel, mesh=_mesh,
    out_type=jax.ShapeDtypeStruct((2, NP, H), jnp.float32),
    scratch_types=[
        pltpu.VMEM((CH, CK), jnp.int32),
        pltpu.VMEM((2, HB, CK), jnp.int32),
        pltpu.VMEM((2, CK, H), jnp.float32),
        pltpu.VMEM_SHARED((NP, H), jnp.float32),
        pltpu.SemaphoreType.DMA,
        pltpu.SemaphoreType.DMA,
        pltpu.SemaphoreType.DMA,
    ])
def _agg_sc(y_hbm, srcr_hbm, dstr_hbm, zerosh_hbm, out_hbm,
            sidx, dring, gbuf, agg_sh, gs0, gs1, dsem):
    c = lax.axis_index("c")
    s = lax.axis_index("s")
    wid = s * 2 + c
    pltpu.sync_copy(zerosh_hbm, agg_sh.at[pl.ds(s * RPS, RPS)])
    pltpu.sync_copy(srcr_hbm.at[wid], sidx)
    pltpu.sync_copy(dstr_hbm.at[wid, pl.ds(0, HB)], dring.at[0])
    pltpu.async_copy(dstr_hbm.at[wid, pl.ds(HB, HB)], dring.at[1], dsem)
    plsc.subcore_barrier()

    pltpu.async_copy(y_hbm.at[sidx.at[0]], gbuf.at[0], gs0)

    for blk in range(NBLK):
        if blk > 0:
            pltpu.make_async_copy(dstr_hbm.at[wid, pl.ds(0, HB)],
                                  dring.at[blk % 2], dsem).wait()
        base = blk * HB

        @pl.loop(base, base + HB, step=2)
        def _(j):
            pltpu.async_copy(y_hbm.at[sidx.at[j + 1]], gbuf.at[1], gs1)
            pltpu.make_async_copy(y_hbm.at[sidx.at[j]], gbuf.at[0], gs0).wait()
            pltpu.sync_copy(gbuf.at[0],
                            agg_sh.at[dring.at[blk % 2, j - base]], add=True)

            @pl.when(j + 2 < CH)
            def _():
                pltpu.async_copy(y_hbm.at[sidx.at[j + 2]], gbuf.at[0], gs0)

            pltpu.make_async_copy(y_hbm.at[sidx.at[j + 1]], gbuf.at[1],
                                  gs1).wait()
            pltpu.sync_copy(gbuf.at[1],
                            agg_sh.at[dring.at[blk % 2, j + 1 - base]],
                            add=True)

        if blk + 2 < NBLK:
            pltpu.async_copy(dstr_hbm.at[wid, pl.ds((blk + 2) * HB, HB)],
                             dring.at[blk % 2], dsem)

    plsc.subcore_barrier()
    pltpu.sync_copy(agg_sh.at[pl.ds(s * RPS, RPS)],
                    out_hbm.at[c, pl.ds(s * RPS, RPS)])


def _mm_body(x_ref, w_ref, o_ref):
    o_ref[...] = jnp.dot(x_ref[...], w_ref[...],
                         preferred_element_type=jnp.float32)


def _matmul(x, w):
    m, k = x.shape
    n = w.shape[1]
    return pl.pallas_call(
        _mm_body,
        grid=(m // BM,),
        in_specs=[pl.BlockSpec((BM, k), lambda i: (i, 0)),
                  pl.BlockSpec((k, n), lambda i: (0, 0))],
        out_specs=pl.BlockSpec((BM, n), lambda i: (i, 0)),
        out_shape=jax.ShapeDtypeStruct((m, n), jnp.float32),
    )(x, w)


def _scale_body(xw_ref, hist_ref, y_ref, dinv_ref):
    deg = hist_ref[0, :, 0:1] + hist_ref[1, :, 0:1] + 1.0
    dinv = lax.rsqrt(deg)
    dinv_ref[...] = dinv
    y_ref[...] = xw_ref[...] * dinv


def _scale(xw, hist):
    return pl.pallas_call(
        _scale_body,
        grid=(NB,),
        in_specs=[pl.BlockSpec((BM, H), lambda i: (i, 0)),
                  pl.BlockSpec((2, BM, H), lambda i: (0, i, 0))],
        out_specs=[pl.BlockSpec((BM, H), lambda i: (i, 0)),
                   pl.BlockSpec((BM, 1), lambda i: (i, 0))],
        out_shape=[jax.ShapeDtypeStruct((NP, H), jnp.float32),
                   jax.ShapeDtypeStruct((NP, 1), jnp.float32)],
    )(xw, hist)


def _mid_body(p_ref, y_ref, dinv_ref, b_ref, w_ref, o_ref):
    dinv = dinv_ref[...]
    h = jnp.maximum(dinv * (p_ref[0] + p_ref[1] + y_ref[...]) + b_ref[...], 0.0)
    o_ref[...] = jnp.dot(h, w_ref[...],
                         preferred_element_type=jnp.float32) * dinv


def _mid(p, y, dinv, b, w):
    return pl.pallas_call(
        _mid_body,
        grid=(NB,),
        in_specs=[pl.BlockSpec((2, BM, H), lambda i: (0, i, 0)),
                  pl.BlockSpec((BM, H), lambda i: (i, 0)),
                  pl.BlockSpec((BM, 1), lambda i: (i, 0)),
                  pl.BlockSpec((1, H), lambda i: (0, 0)),
                  pl.BlockSpec((H, H), lambda i: (0, 0))],
        out_specs=pl.BlockSpec((BM, H), lambda i: (i, 0)),
        out_shape=jax.ShapeDtypeStruct((NP, H), jnp.float32),
    )(p, y, dinv, b, w)


def _final_body(q_ref, y_ref, dinv_ref, b_ref, batch_ref, wl_ref, bl_ref,
                o_ref, summ, cnt):
    i = pl.program_id(0)

    @pl.when(i == 0)
    def _():
        summ[...] = jnp.zeros_like(summ)
        cnt[...] = jnp.zeros_like(cnt)

    dinv = dinv_ref[...]
    h = jnp.maximum(dinv * (q_ref[0] + q_ref[1] + y_ref[...]) + b_ref[...], 0.0)
    gid = lax.broadcasted_iota(jnp.int32, (G, BM), 0)
    oht = (batch_ref[...][None, :] == gid).astype(jnp.float32)
    summ[...] += jnp.dot(oht, h, preferred_element_type=jnp.float32)
    cnt[...] += jnp.sum(oht, axis=1, keepdims=True)

    @pl.when(i == NB - 1)
    def _():
        pooled = summ[...] / jnp.maximum(cnt[...], 1.0)
        logits = jnp.dot(pooled, wl_ref[...],
                         preferred_element_type=jnp.float32) + bl_ref[...]
        m = jnp.max(logits, axis=1, keepdims=True)
        lse = m + jnp.log(jnp.sum(jnp.exp(logits - m), axis=1, keepdims=True))
        o_ref[...] = logits - lse


def _final(q, y, dinv, b, batchp, wl, bl):
    return pl.pallas_call(
        _final_body,
        grid=(NB,),
        in_specs=[pl.BlockSpec((2, BM, H), lambda i: (0, i, 0)),
                  pl.BlockSpec((BM, H), lambda i: (i, 0)),
                  pl.BlockSpec((BM, 1), lambda i: (i, 0)),
                  pl.BlockSpec((1, H), lambda i: (0, 0)),
                  pl.BlockSpec((BM,), lambda i: (i,)),
                  pl.BlockSpec((H, C), lambda i: (0, 0)),
                  pl.BlockSpec((1, C), lambda i: (0, 0))],
        out_specs=pl.BlockSpec((G, C), lambda i: (0, 0)),
        out_shape=jax.ShapeDtypeStruct((G, C), jnp.float32),
        scratch_shapes=[pltpu.VMEM((G, H), jnp.float32),
                        pltpu.VMEM((G, 1), jnp.float32)],
    )(q, y, dinv, b, batchp, wl, bl)


def kernel(x, edge_index, batch, W1, b1, W2, b2, Wlin, blin):
    pad = EP - E
    pad_idx = N + (jnp.arange(pad, dtype=jnp.int32) % (NP - N))
    srcp = jnp.concatenate([edge_index[0], pad_idx]).reshape(NT, CH, CK)
    dstp = jnp.concatenate([edge_index[1], pad_idx]).reshape(NT, CH, CK)
    xp = jnp.pad(x, ((0, NP - N), (0, 0)))
    batchp = jnp.pad(batch, (0, NP - N), constant_values=-1)
    onesh = jnp.ones((CK, H), jnp.float32)
    zerosh = jnp.zeros((RPS, H), jnp.float32)

    hist = _hist_sc(dstp, zerosh, onesh)
    xw1 = _matmul(xp, W1)
    y1, dinv = _scale(xw1, hist)
    p = _agg_sc(y1, srcp, dstp, zerosh)
    y2 = _mid(p, y1, dinv, b1.reshape(1, H), W2)
    q = _agg_sc(y2, srcp, dstp, zerosh)
    return _final(q, y2, dinv, b2.reshape(1, H), batchp, Wlin,
                  blin.reshape(1, C))
